# baseline (speedup 1.0000x reference)
import functools
import numpy as np
import jax
import jax.numpy as jnp
from jax import lax
from jax.experimental import pallas as pl
from jax.experimental.pallas import tpu as pltpu
from jax.experimental.pallas import tpu_sc as plsc

_D = 256
_SEQ = 200
_SCALE = np.float32(np.sqrt(np.float32(_D)))
_NC = 2
_NS = 16
_NW = _NC * _NS
_R = 128
_LANES = 16


def _pos_encoding(length, depth):
    depth_h = depth / 2
    positions = np.arange(length)[:, np.newaxis]
    depths = np.arange(depth_h)[np.newaxis, :] / depth_h
    angle_rates = 1 / 10000 ** depths
    angle_rads = positions * angle_rates
    return np.concatenate([np.sin(angle_rads), np.cos(angle_rads)],
                          axis=-1).astype(np.float32)


@functools.partial(jax.jit, static_argnames=("chunks_per_tile",))
def _sc_embed(idx3, table, pos, *, chunks_per_tile):
    rows_per_tile = chunks_per_tile * _R
    total = _NW * rows_per_tile
    mesh = plsc.VectorSubcoreMesh(core_axis_name="c", subcore_axis_name="s")

    @functools.partial(
        pl.kernel,
        mesh=mesh,
        out_type=jax.ShapeDtypeStruct((total, _D), jnp.float32),
        scratch_types=[
            pltpu.VMEM((chunks_per_tile, _R), jnp.int32),
            pltpu.VMEM((_SEQ, _D), jnp.float32),
            pltpu.VMEM((_R, _D), jnp.float32),
            pltpu.VMEM((_R, _D), jnp.float32),
            pltpu.SemaphoreType.DMA,
            pltpu.SemaphoreType.DMA,
            pltpu.SemaphoreType.DMA,
            pltpu.SemaphoreType.DMA,
        ],
    )
    def k(idx_hbm, table_hbm, pos_hbm, out_hbm,
          idx_v, pos_v, buf_a, buf_b, gs_a, gs_b, os_a, os_b):
        wid = lax.axis_index("s") * _NC + lax.axis_index("c")
        base = wid * rows_per_tile
        pltpu.sync_copy(idx_hbm.at[wid], idx_v)
        pltpu.sync_copy(pos_hbm, pos_v)

        def gather_start(c, buf, sem):
            pltpu.async_copy(table_hbm.at[idx_v.at[c]], buf, sem)

        def gather_wait(c, buf, sem):
            pltpu.make_async_copy(table_hbm.at[idx_v.at[c]], buf, sem).wait()

        def out_start(c, buf, sem):
            pltpu.async_copy(buf, out_hbm.at[pl.ds(base + c * _R, _R)], sem)

        def out_wait(c, buf, sem):
            pltpu.make_async_copy(
                buf, out_hbm.at[pl.ds(base + c * _R, _R)], sem).wait()

        def compute(buf, c):
            j0 = lax.rem(c * _R, _SEQ)

            def body(r, carry):
                row = j0 + r
                row = jnp.where(row >= _SEQ, row - _SEQ, row)
                for kk in range(_D // _LANES):
                    sl = pl.ds(kk * _LANES, _LANES)
                    g = buf[r, sl]
                    p = pos_v[row, sl]
                    buf[r, sl] = g * _SCALE + p
                return carry
            lax.fori_loop(0, _R, body, 0)

        gather_start(0, buf_a, gs_a)

        def pair(g, carry):
            c0 = 2 * g
            gather_wait(c0, buf_a, gs_a)

            @pl.when(g > 0)
            def _():
                out_wait(c0 - 1, buf_b, os_b)

            gather_start(c0 + 1, buf_b, gs_b)
            compute(buf_a, c0)
            out_start(c0, buf_a, os_a)

            gather_wait(c0 + 1, buf_b, gs_b)
            out_wait(c0, buf_a, os_a)

            @pl.when(g < chunks_per_tile // 2 - 1)
            def _():
                gather_start(c0 + 2, buf_a, gs_a)

            compute(buf_b, c0 + 1)
            out_start(c0 + 1, buf_b, os_b)
            return carry

        lax.fori_loop(0, chunks_per_tile // 2, pair, 0)
        out_wait(chunks_per_tile - 1, buf_b, os_b)

    return k(idx3, table, pos)


def kernel(input, table):
    b, s = input.shape
    d = table.shape[1]
    assert d == _D and s == _SEQ and (b * s) % (_NW * 2 * _R) == 0, (b, s, d)
    chunks_per_tile = (b * s) // (_NW * _R)
    idx3 = jnp.asarray(input, jnp.int32).reshape(_NW, chunks_per_tile, _R)
    pos = jnp.asarray(_pos_encoding(_SEQ, _D))
    out = _sc_embed(idx3, table, pos, chunks_per_tile=chunks_per_tile)
    return out.reshape(b, s, _D)


# speedup vs baseline: 2.5912x; 2.5912x over previous
import functools
import numpy as np
import jax
import jax.numpy as jnp
from jax import lax
from jax.experimental import pallas as pl
from jax.experimental.pallas import tpu as pltpu
from jax.experimental.pallas import tpu_sc as plsc

_D = 256
_SEQ = 200
_SCALE = np.float32(np.sqrt(np.float32(_D)))
_NC = 2
_NS = 16
_NW = _NC * _NS
_R = 128
_LANES = 16


def _pos_encoding(length, depth):
    depth_h = depth / 2
    positions = np.arange(length)[:, np.newaxis]
    depths = np.arange(depth_h)[np.newaxis, :] / depth_h
    angle_rates = 1 / 10000 ** depths
    angle_rads = positions * angle_rates
    return np.concatenate([np.sin(angle_rads), np.cos(angle_rads)],
                          axis=-1).astype(np.float32)


@functools.partial(jax.jit, static_argnames=("chunks_per_tile",))
def _sc_embed(idx3, table, pos, *, chunks_per_tile):
    rows_per_tile = chunks_per_tile * _R
    total = _NW * rows_per_tile
    mesh = plsc.VectorSubcoreMesh(core_axis_name="c", subcore_axis_name="s")

    @functools.partial(
        pl.kernel,
        mesh=mesh,
        out_type=jax.ShapeDtypeStruct((total, _D), jnp.float32),
        scratch_types=[
            pltpu.VMEM((chunks_per_tile, _R), jnp.int32),
            pltpu.VMEM((_SEQ, _D), jnp.float32),
            pltpu.VMEM((_R, _D), jnp.float32),
            pltpu.VMEM((_R, _D), jnp.float32),
            pltpu.SemaphoreType.DMA,
            pltpu.SemaphoreType.DMA,
            pltpu.SemaphoreType.DMA,
            pltpu.SemaphoreType.DMA,
        ],
    )
    def k(idx_hbm, table_hbm, pos_hbm, out_hbm,
          idx_v, pos_v, buf_a, buf_b, gs_a, gs_b, os_a, os_b):
        wid = lax.axis_index("s") * _NC + lax.axis_index("c")
        base = wid * rows_per_tile
        pltpu.sync_copy(idx_hbm.at[wid], idx_v)
        pltpu.sync_copy(pos_hbm, pos_v)

        def gather_start(c, buf, sem):
            pltpu.async_copy(table_hbm.at[idx_v.at[c]], buf, sem)

        def gather_wait(c, buf, sem):
            pltpu.make_async_copy(table_hbm.at[idx_v.at[c]], buf, sem).wait()

        def out_start(c, buf, sem):
            pltpu.async_copy(buf, out_hbm.at[pl.ds(base + c * _R, _R)], sem)

        def out_wait(c, buf, sem):
            pltpu.make_async_copy(
                buf, out_hbm.at[pl.ds(base + c * _R, _R)], sem).wait()

        def compute(buf, c):
            j0 = lax.rem(c * _R, _SEQ)

            def body(r, carry):
                row = j0 + r
                row = jnp.where(row >= _SEQ, row - _SEQ, row)
                nk = _D // _LANES
                sls = [pl.ds(kk * _LANES, _LANES) for kk in range(nk)]
                gs = [buf[r, sl] for sl in sls]
                ps = [pos_v[row, sl] for sl in sls]
                for kk in range(nk):
                    buf[r, sls[kk]] = gs[kk] * _SCALE + ps[kk]
                return carry
            lax.fori_loop(0, _R, body, 0)

        gather_start(0, buf_a, gs_a)

        def pair(g, carry):
            c0 = 2 * g
            gather_wait(c0, buf_a, gs_a)

            @pl.when(g > 0)
            def _():
                out_wait(c0 - 1, buf_b, os_b)

            gather_start(c0 + 1, buf_b, gs_b)
            compute(buf_a, c0)
            out_start(c0, buf_a, os_a)

            gather_wait(c0 + 1, buf_b, gs_b)
            out_wait(c0, buf_a, os_a)

            @pl.when(g < chunks_per_tile // 2 - 1)
            def _():
                gather_start(c0 + 2, buf_a, gs_a)

            compute(buf_b, c0 + 1)
            out_start(c0 + 1, buf_b, os_b)
            return carry

        lax.fori_loop(0, chunks_per_tile // 2, pair, 0)
        out_wait(chunks_per_tile - 1, buf_b, os_b)

    return k(idx3, table, pos)


def kernel(input, table):
    b, s = input.shape
    d = table.shape[1]
    assert d == _D and s == _SEQ and (b * s) % (_NW * 2 * _R) == 0, (b, s, d)
    chunks_per_tile = (b * s) // (_NW * _R)
    idx3 = jnp.asarray(input, jnp.int32).reshape(_NW, chunks_per_tile, _R)
    pos = jnp.asarray(_pos_encoding(_SEQ, _D))
    out = _sc_embed(idx3, table, pos, chunks_per_tile=chunks_per_tile)
    return out.reshape(b, s, _D)
